# Initial kernel scaffold; baseline (speedup 1.0000x reference)
import functools

import jax
import jax.numpy as jnp
from jax import lax
from jax.experimental import pallas as pl
from jax.experimental.pallas import tpu as pltpu
from jax.experimental.pallas import tpu_sc as plsc

N = 10000
E = 320000
G = 256
H = 128
AF = 128
LAYERS = 4

NC = 2
NS = 16
NW = NC * NS
C = 128
NCH = 79
EWP = NCH * C
EP = NW * EWP
NP = 10240
WD = 144
NB = 512
NBLK = NP // NB
TE = 1024
NEB = EP // TE

_f32 = jnp.float32


def _sc_mesh():
    return plsc.VectorSubcoreMesh(
        core_axis_name="c", subcore_axis_name="s", num_cores=NC, num_subcores=NS
    )


def _gather_body(td_h, ts_h, dst_h, src_h, od_h, os_h, idxd, rowd, idxs, rows,
                 semd, sems):
    wid = lax.axis_index("s") * NC + lax.axis_index("c")
    base = wid * EWP

    @pl.loop(0, NCH)
    def _chunk(j):
        off = base + j * C
        pltpu.sync_copy(dst_h.at[pl.ds(off, C)], idxd)
        cpd = pltpu.async_copy(td_h.at[idxd], rowd, semd)
        pltpu.sync_copy(src_h.at[pl.ds(off, C)], idxs)
        cps = pltpu.async_copy(ts_h.at[idxs], rows, sems)
        cpd.wait()
        pltpu.sync_copy(rowd, od_h.at[pl.ds(off, C)])
        cps.wait()
        pltpu.sync_copy(rows, os_h.at[pl.ds(off, C)])


def _sc_gather(td, ts, dstp, srcp):
    return pl.kernel(
        _gather_body,
        out_type=(
            jax.ShapeDtypeStruct((EP, WD), _f32),
            jax.ShapeDtypeStruct((EP, WD), _f32),
        ),
        mesh=_sc_mesh(),
        scratch_types=[
            pltpu.VMEM((C,), jnp.int32),
            pltpu.VMEM((C, WD), _f32),
            pltpu.VMEM((C,), jnp.int32),
            pltpu.VMEM((C, WD), _f32),
            pltpu.SemaphoreType.DMA,
            pltpu.SemaphoreType.DMA,
        ],
    )(td, ts, dstp, srcp)


_ROWS_PER_TILE = NP // NS
_ZCH = _ROWS_PER_TILE // C


def _scatter_body(vals_h, dst_h, out_h, idxv, vbuf, acc, semz):
    cid = lax.axis_index("c")
    sid = lax.axis_index("s")
    wid = sid * NC + cid
    base = wid * EWP
    row0 = sid * _ROWS_PER_TILE

    zero16 = jnp.zeros((16,), _f32)

    @pl.loop(0, C)
    def _zrow(i):
        for c in range(WD // 16):
            vbuf[i, pl.ds(c * 16, 16)] = zero16

    @pl.loop(0, _ZCH)
    def _zacc(k):
        pltpu.sync_copy(vbuf, acc.at[pl.ds(row0 + k * C, C)])

    plsc.subcore_barrier()

    @pl.loop(0, NCH)
    def _chunk(j):
        off = base + j * C
        pltpu.sync_copy(dst_h.at[pl.ds(off, C)], idxv)
        pltpu.sync_copy(vals_h.at[pl.ds(off, C)], vbuf)
        pltpu.sync_copy(vbuf, acc.at[idxv], add=True)

    plsc.subcore_barrier()

    @pl.loop(0, _ZCH)
    def _wout(k):
        pltpu.sync_copy(acc.at[pl.ds(row0 + k * C, C)],
                        out_h.at[cid, pl.ds(row0 + k * C, C)])


def _sc_scatter(vals, dstp):
    return pl.kernel(
        _scatter_body,
        out_type=jax.ShapeDtypeStruct((NC, NP, WD), _f32),
        mesh=_sc_mesh(),
        scratch_types=[
            pltpu.VMEM((C,), jnp.int32),
            pltpu.VMEM((C, WD), _f32),
            pltpu.VMEM_SHARED((NP, WD), _f32),
            pltpu.SemaphoreType.DMA,
        ],
    )(vals, dstp)


def _com_first_body(c_ref, nd_ref, com_ref, acc):
    i = pl.program_id(0)

    @pl.when(i == 0)
    def _():
        acc[...] = jnp.zeros_like(acc)

    nd = nd_ref[...]
    gidf = nd[:, 0:1]
    m = nd[:, 1:2]
    cc = c_ref[...][:, 0:3]
    oh = (gidf == lax.broadcasted_iota(_f32, (NB, G), 1)).astype(_f32)
    vals = jnp.concatenate([m * cc, m, jnp.zeros((NB, 4), _f32)], axis=1)
    acc[...] += lax.dot_general(oh, vals, (((0,), (0,)), ((), ())),
                                preferred_element_type=_f32)

    @pl.when(i == NBLK - 1)
    def _():
        a = acc[...]
        com = a[:, 0:3] / jnp.clip(a[:, 3:4], 1e-6, None)
        com_ref[...] = jnp.concatenate([com, jnp.zeros((G, 5), _f32)], axis=1)


def _com_first(coords8, nodes8):
    return pl.pallas_call(
        _com_first_body,
        grid=(NBLK,),
        in_specs=[
            pl.BlockSpec((NB, 8), lambda i: (i, 0)),
            pl.BlockSpec((NB, 8), lambda i: (i, 0)),
        ],
        out_specs=pl.BlockSpec((G, 8), lambda i: (0, 0)),
        out_shape=jax.ShapeDtypeStruct((G, 8), _f32),
        scratch_shapes=[pltpu.VMEM((G, 8), _f32)],
    )(coords8, nodes8)


def _com_next_body(cz_ref, aggA_ref, aggB_ref, nd_ref, com_ref, cnew_ref, acc):
    i = pl.program_id(0)

    @pl.when(i == 0)
    def _():
        acc[...] = jnp.zeros_like(acc)

    nd = nd_ref[...]
    gidf = nd[:, 0:1]
    m = nd[:, 1:2]
    cnew = (cz_ref[...][:, 0:3]
            + aggA_ref[...][:, 128:131] + aggB_ref[...][:, 128:131])
    cnew_ref[...] = jnp.concatenate([cnew, jnp.zeros((NB, 5), _f32)], axis=1)
    oh = (gidf == lax.broadcasted_iota(_f32, (NB, G), 1)).astype(_f32)
    vals = jnp.concatenate([m * cnew, m, jnp.zeros((NB, 4), _f32)], axis=1)
    acc[...] += lax.dot_general(oh, vals, (((0,), (0,)), ((), ())),
                                preferred_element_type=_f32)

    @pl.when(i == NBLK - 1)
    def _():
        a = acc[...]
        com = a[:, 0:3] / jnp.clip(a[:, 3:4], 1e-6, None)
        com_ref[...] = jnp.concatenate([com, jnp.zeros((G, 5), _f32)], axis=1)


def _com_next(cz8, aggA, aggB, nodes8):
    return pl.pallas_call(
        _com_next_body,
        grid=(NBLK,),
        in_specs=[
            pl.BlockSpec((NB, 8), lambda i: (i, 0)),
            pl.BlockSpec((NB, WD), lambda i: (i, 0)),
            pl.BlockSpec((NB, WD), lambda i: (i, 0)),
            pl.BlockSpec((NB, 8), lambda i: (i, 0)),
        ],
        out_specs=[
            pl.BlockSpec((G, 8), lambda i: (0, 0)),
            pl.BlockSpec((NB, 8), lambda i: (i, 0)),
        ],
        out_shape=[
            jax.ShapeDtypeStruct((G, 8), _f32),
            jax.ShapeDtypeStruct((NP, 8), _f32),
        ],
        scratch_shapes=[pltpu.VMEM((G, 8), _f32)],
    )(cz8, aggA, aggB, nodes8)


def _init_body(nd_ref, craw_ref, f16_ref, com_ref, embp_ref, pWt_ref, pWb_ref,
               pb_ref, Wd_ref, Ws_ref, eb1_ref, h_ref, td_ref, ts_ref, cz_ref):
    nd = nd_ref[...]
    gidf = nd[:, 0:1]
    anf = nd[:, 2:3]
    ohA = (anf == lax.broadcasted_iota(_f32, (NB, 128), 1)).astype(_f32)
    tbl = jnp.dot(embp_ref[...], pWt_ref[...], preferred_element_type=_f32)
    h = (jnp.dot(ohA, tbl, preferred_element_type=_f32)
         + jnp.dot(f16_ref[...], pWb_ref[...], preferred_element_type=_f32)
         + pb_ref[...])
    ohG = (gidf == lax.broadcasted_iota(_f32, (NB, G), 1)).astype(_f32)
    craw = craw_ref[...][:, 0:3]
    cz = craw - jnp.dot(ohG, com_ref[...], preferred_element_type=_f32)[:, 0:3]
    h_ref[...] = h
    zpad = jnp.zeros((NB, 10), _f32)
    td_ref[...] = jnp.concatenate(
        [jnp.dot(h, Wd_ref[...], preferred_element_type=_f32) + eb1_ref[...],
         cz, craw, zpad], axis=1)
    ts_ref[...] = jnp.concatenate(
        [jnp.dot(h, Ws_ref[...], preferred_element_type=_f32),
         -cz, -craw, zpad], axis=1)
    cz_ref[...] = jnp.concatenate([cz, jnp.zeros((NB, 5), _f32)], axis=1)


def _init(nodes8, coords8, feat16, com0, embp, pWt, pWb, pb, Wd, Ws, eb1):
    full = lambda shp: pl.BlockSpec(shp, lambda i: tuple(0 for _ in shp))
    return pl.pallas_call(
        _init_body,
        grid=(NBLK,),
        in_specs=[
            pl.BlockSpec((NB, 8), lambda i: (i, 0)),
            pl.BlockSpec((NB, 8), lambda i: (i, 0)),
            pl.BlockSpec((NB, 16), lambda i: (i, 0)),
            full((G, 8)),
            full((128, 128)),
            full((128, 128)),
            full((16, 128)),
            full((1, 128)),
            full((128, 128)),
            full((128, 128)),
            full((1, 128)),
        ],
        out_specs=[
            pl.BlockSpec((NB, 128), lambda i: (i, 0)),
            pl.BlockSpec((NB, WD), lambda i: (i, 0)),
            pl.BlockSpec((NB, WD), lambda i: (i, 0)),
            pl.BlockSpec((NB, 8), lambda i: (i, 0)),
        ],
        out_shape=[
            jax.ShapeDtypeStruct((NP, 128), _f32),
            jax.ShapeDtypeStruct((NP, WD), _f32),
            jax.ShapeDtypeStruct((NP, WD), _f32),
            jax.ShapeDtypeStruct((NP, 8), _f32),
        ],
    )(nodes8, coords8, feat16, com0, embp, pWt, pWb, pb, Wd, Ws, eb1)


def _edge_core(sd, a, wa, eW2, eb2, xW1, xb1, xW2p):
    hsum = sd[:, 0:128]
    diff = sd[:, 128:131]
    v = hsum + a * wa
    m1 = jax.nn.silu(v)
    m = jax.nn.silu(jnp.dot(m1, eW2, preferred_element_type=_f32) + eb2)
    t = jax.nn.silu(jnp.dot(m, xW1, preferred_element_type=_f32) + xb1)
    w = jnp.dot(t, xW2p, preferred_element_type=_f32)[:, 0:1]
    dist = jnp.sqrt(jnp.sum(diff * diff, axis=1, keepdims=True) + 1e-8)
    trans = diff / (dist + 1.0) * w
    return jnp.concatenate([m, trans, jnp.zeros((sd.shape[0], 13), _f32)], 1)


def _edge_a1_body(gd_ref, gs_ref, wa_ref, eW2_ref, eb2_ref, xW1_ref, xb1_ref,
                  xW2_ref, vals_ref, a_ref):
    sd = gd_ref[...] + gs_ref[...]
    ar = sd[:, 131:134]
    a = jnp.sum(ar * ar, axis=1, keepdims=True)
    vals_ref[...] = _edge_core(sd, a, wa_ref[...], eW2_ref[...], eb2_ref[...],
                               xW1_ref[...], xb1_ref[...], xW2_ref[...])
    a_ref[...] = a


def _edge_a2_body(gd_ref, gs_ref, a_in_ref, wa_ref, eW2_ref, eb2_ref, xW1_ref,
                  xb1_ref, xW2_ref, vals_ref):
    sd = gd_ref[...] + gs_ref[...]
    a = a_in_ref[...]
    vals_ref[...] = _edge_core(sd, a, wa_ref[...], eW2_ref[...], eb2_ref[...],
                               xW1_ref[...], xb1_ref[...], xW2_ref[...])


def _edge_weight_specs():
    full = lambda shp: pl.BlockSpec(shp, lambda i: tuple(0 for _ in shp))
    return [full((1, 128)), full((128, 128)), full((1, 128)),
            full((128, 128)), full((1, 128)), full((128, 8))]


def _edge_a1(gd, gs, wa, eW2, eb2, xW1, xb1, xW2p):
    return pl.pallas_call(
        _edge_a1_body,
        grid=(NEB,),
        in_specs=[
            pl.BlockSpec((TE, WD), lambda i: (i, 0)),
            pl.BlockSpec((TE, WD), lambda i: (i, 0)),
        ] + _edge_weight_specs(),
        out_specs=[
            pl.BlockSpec((TE, WD), lambda i: (i, 0)),
            pl.BlockSpec((TE, 1), lambda i: (i, 0)),
        ],
        out_shape=[
            jax.ShapeDtypeStruct((EP, WD), _f32),
            jax.ShapeDtypeStruct((EP, 1), _f32),
        ],
    )(gd, gs, wa, eW2, eb2, xW1, xb1, xW2p)


def _edge_a2(gd, gs, a_edges, wa, eW2, eb2, xW1, xb1, xW2p):
    return pl.pallas_call(
        _edge_a2_body,
        grid=(NEB,),
        in_specs=[
            pl.BlockSpec((TE, WD), lambda i: (i, 0)),
            pl.BlockSpec((TE, WD), lambda i: (i, 0)),
            pl.BlockSpec((TE, 1), lambda i: (i, 0)),
        ] + _edge_weight_specs(),
        out_specs=pl.BlockSpec((TE, WD), lambda i: (i, 0)),
        out_shape=jax.ShapeDtypeStruct((EP, WD), _f32),
    )(gd, gs, a_edges, wa, eW2, eb2, xW1, xb1, xW2p)


def _b2_body(h_ref, aggA_ref, aggB_ref, cnew_ref, nd_ref, com_ref, hW1a_ref,
             hW1b_ref, hb1_ref, hW2_ref, hb2_ref, Wd_ref, Ws_ref, eb1_ref,
             hn_ref, td_ref, ts_ref, cz_ref):
    h = h_ref[...]
    aggm = aggA_ref[...][:, 0:128] + aggB_ref[...][:, 0:128]
    u = jax.nn.silu(jnp.dot(h, hW1a_ref[...], preferred_element_type=_f32)
                    + jnp.dot(aggm, hW1b_ref[...], preferred_element_type=_f32)
                    + hb1_ref[...])
    hn = h + jnp.dot(u, hW2_ref[...], preferred_element_type=_f32) + hb2_ref[...]
    nd = nd_ref[...]
    gidf = nd[:, 0:1]
    ohG = (gidf == lax.broadcasted_iota(_f32, (NB, G), 1)).astype(_f32)
    cz = (cnew_ref[...][:, 0:3]
          - jnp.dot(ohG, com_ref[...], preferred_element_type=_f32)[:, 0:3])
    hn_ref[...] = hn
    zpad = jnp.zeros((NB, 13), _f32)
    td_ref[...] = jnp.concatenate(
        [jnp.dot(hn, Wd_ref[...], preferred_element_type=_f32) + eb1_ref[...],
         cz, zpad], axis=1)
    ts_ref[...] = jnp.concatenate(
        [jnp.dot(hn, Ws_ref[...], preferred_element_type=_f32), -cz, zpad],
        axis=1)
    cz_ref[...] = jnp.concatenate([cz, jnp.zeros((NB, 5), _f32)], axis=1)


def _b2(h, aggA, aggB, cnew8, nodes8, com, hW1a, hW1b, hb1, hW2, hb2, Wd, Ws,
        eb1):
    full = lambda shp: pl.BlockSpec(shp, lambda i: tuple(0 for _ in shp))
    return pl.pallas_call(
        _b2_body,
        grid=(NBLK,),
        in_specs=[
            pl.BlockSpec((NB, 128), lambda i: (i, 0)),
            pl.BlockSpec((NB, WD), lambda i: (i, 0)),
            pl.BlockSpec((NB, WD), lambda i: (i, 0)),
            pl.BlockSpec((NB, 8), lambda i: (i, 0)),
            pl.BlockSpec((NB, 8), lambda i: (i, 0)),
            full((G, 8)),
            full((128, 128)), full((128, 128)), full((1, 128)),
            full((128, 128)), full((1, 128)),
            full((128, 128)), full((128, 128)), full((1, 128)),
        ],
        out_specs=[
            pl.BlockSpec((NB, 128), lambda i: (i, 0)),
            pl.BlockSpec((NB, WD), lambda i: (i, 0)),
            pl.BlockSpec((NB, WD), lambda i: (i, 0)),
            pl.BlockSpec((NB, 8), lambda i: (i, 0)),
        ],
        out_shape=[
            jax.ShapeDtypeStruct((NP, 128), _f32),
            jax.ShapeDtypeStruct((NP, WD), _f32),
            jax.ShapeDtypeStruct((NP, WD), _f32),
            jax.ShapeDtypeStruct((NP, 8), _f32),
        ],
    )(h, aggA, aggB, cnew8, nodes8, com, hW1a, hW1b, hb1, hW2, hb2, Wd, Ws, eb1)


def _final_body(cnew_ref, nd_ref, com_ref, out_ref):
    nd = nd_ref[...]
    gidf = nd[:, 0:1]
    ohG = (gidf == lax.broadcasted_iota(_f32, (NB, G), 1)).astype(_f32)
    cz = (cnew_ref[...][:, 0:3]
          - jnp.dot(ohG, com_ref[...], preferred_element_type=_f32)[:, 0:3])
    out_ref[...] = jnp.concatenate([cz, jnp.zeros((NB, 5), _f32)], axis=1)


def _final(cnew8, nodes8, com):
    full = lambda shp: pl.BlockSpec(shp, lambda i: tuple(0 for _ in shp))
    return pl.pallas_call(
        _final_body,
        grid=(NBLK,),
        in_specs=[
            pl.BlockSpec((NB, 8), lambda i: (i, 0)),
            pl.BlockSpec((NB, 8), lambda i: (i, 0)),
            full((G, 8)),
        ],
        out_specs=pl.BlockSpec((NB, 8), lambda i: (i, 0)),
        out_shape=jax.ShapeDtypeStruct((NP, 8), _f32),
    )(cnew8, nodes8, com)


def _pad_edges(idx):
    per = idx.reshape(NW, E // NW)
    per = jnp.pad(per, ((0, 0), (0, EWP - E // NW)), constant_values=N)
    return per.reshape(EP).astype(jnp.int32)


def kernel(temb, masses, masses_normalized, cond_labels, cond_mask, moments,
           coords, emb, proj_W, proj_b, eW1, eb1, eW2, eb2, xW1, xb1, xW2,
           hW1, hb1, hW2, hb2, atom_nums, edge_index, node_graph_idx):
    srcp = _pad_edges(edge_index[0])
    dstp = _pad_edges(edge_index[1])

    gidf = node_graph_idx.astype(_f32)[:, None]
    anf = atom_nums.astype(_f32)[:, None]
    nodes8 = jnp.pad(
        jnp.concatenate([gidf, masses, anf], axis=1), ((0, NP - N), (0, 5)))
    coords8 = jnp.pad(coords, ((0, NP - N), (0, 5)))
    feat16 = jnp.pad(
        jnp.concatenate([temb, masses, masses_normalized, cond_labels,
                         cond_mask, moments], axis=1), ((0, NP - N), (0, 4)))
    embp = jnp.pad(emb, ((0, 128 - emb.shape[0]), (0, 0)))
    pWt = proj_W[:AF]
    pWb = jnp.pad(proj_W[AF:], ((0, 4), (0, 0)))
    pb = proj_b[None, :]
    xW2p = jnp.pad(xW2, ((0, 0), (0, 0), (0, 7)))

    com0 = _com_first(coords8, nodes8)
    h, td, ts, cz8 = _init(nodes8, coords8, feat16, com0, embp, pWt, pWb, pb,
                           eW1[0, 0:H], eW1[0, H:2 * H], eb1[0][None, :])

    a_edges = None
    out = None
    for i in range(LAYERS):
        gd, gs = _sc_gather(td, ts, dstp, srcp)
        wa = eW1[i, 2 * H:2 * H + 1]
        if i == 0:
            vals, a_edges = _edge_a1(gd, gs, wa, eW2[i], eb2[i][None, :],
                                     xW1[i], xb1[i][None, :], xW2p[i])
        else:
            vals = _edge_a2(gd, gs, a_edges, wa, eW2[i], eb2[i][None, :],
                            xW1[i], xb1[i][None, :], xW2p[i])
        agg = _sc_scatter(vals, dstp)
        aggA, aggB = agg[0], agg[1]
        com, cnew8 = _com_next(cz8, aggA, aggB, nodes8)
        if i + 1 < LAYERS:
            h, td, ts, cz8 = _b2(h, aggA, aggB, cnew8, nodes8, com,
                                 hW1[i, 0:H], hW1[i, H:2 * H],
                                 hb1[i][None, :], hW2[i], hb2[i][None, :],
                                 eW1[i + 1, 0:H], eW1[i + 1, H:2 * H],
                                 eb1[i + 1][None, :])
        else:
            out = _final(cnew8, nodes8, com)
    return out[:N, 0:3]


# speedup vs baseline: 2.4333x; 2.4333x over previous
import jax
import jax.numpy as jnp
from jax import lax
from jax.experimental import pallas as pl
from jax.experimental.pallas import tpu as pltpu
from jax.experimental.pallas import tpu_sc as plsc

N = 10000
E = 320000
G = 256
H = 128
AF = 128
LAYERS = 4

NC = 2
NS = 16
NW = NC * NS
C = 128
NCH = 79
EWP = NCH * C
EP = NW * EWP
NP = 10240
NPG = NP // 16
TW = 256
NB = 512
NBLK = NP // NB
TE = 1024
NEB = EP // TE

_f32 = jnp.float32


def _sc_mesh():
    return plsc.VectorSubcoreMesh(
        core_axis_name="c", subcore_axis_name="s", num_cores=NC, num_subcores=NS
    )


def _gather_body(td_h, ts_h, dst_h, src_h, od_h, os_h, idxd, rowd, idxs, rows,
                 semd, sems):
    wid = lax.axis_index("s") * NC + lax.axis_index("c")
    base = wid * EWP

    @pl.loop(0, NCH)
    def _chunk(j):
        off = base + j * C
        pltpu.sync_copy(dst_h.at[pl.ds(off, C)], idxd)
        cpd = pltpu.async_copy(td_h.at[idxd], rowd, semd)
        pltpu.sync_copy(src_h.at[pl.ds(off, C)], idxs)
        cps = pltpu.async_copy(ts_h.at[idxs], rows, sems)
        cpd.wait()
        pltpu.sync_copy(rowd, od_h.at[pl.ds(off, C)])
        cps.wait()
        pltpu.sync_copy(rows, os_h.at[pl.ds(off, C)])


def _sc_gather(td, ts, dstp, srcp):
    return pl.kernel(
        _gather_body,
        out_type=(
            jax.ShapeDtypeStruct((EP, TW), _f32),
            jax.ShapeDtypeStruct((EP, TW), _f32),
        ),
        mesh=_sc_mesh(),
        scratch_types=[
            pltpu.VMEM((C,), jnp.int32),
            pltpu.VMEM((C, TW), _f32),
            pltpu.VMEM((C,), jnp.int32),
            pltpu.VMEM((C, TW), _f32),
            pltpu.SemaphoreType.DMA,
            pltpu.SemaphoreType.DMA,
        ],
    )(td, ts, dstp, srcp)


_MROWS = NP // NS
_MZCH = _MROWS // C
_TROWS = NPG // NS


def _scatter_body(valm_h, valt_h, dst_h, outm_h, outt_h, idxv, idxt, vbuf,
                  accm, acct):
    cid = lax.axis_index("c")
    sid = lax.axis_index("s")
    wid = sid * NC + cid
    base = wid * EWP
    mrow0 = sid * _MROWS
    trow0 = sid * _TROWS

    zero16 = jnp.zeros((16,), _f32)

    @pl.loop(0, C)
    def _zrow(i):
        for c in range(128 // 16):
            vbuf[i, pl.ds(c * 16, 16)] = zero16

    @pl.loop(0, _MZCH)
    def _zaccm(k):
        pltpu.sync_copy(vbuf, accm.at[pl.ds(mrow0 + k * C, C)])

    pltpu.sync_copy(vbuf.at[pl.ds(0, _TROWS)], acct.at[pl.ds(trow0, _TROWS)])

    plsc.subcore_barrier()

    @pl.loop(0, NCH)
    def _chunk(j):
        off = base + j * C
        pltpu.sync_copy(dst_h.at[pl.ds(off, C)], idxv)
        for g in range(C // 16):
            idxt[pl.ds(g * 16, 16)] = (
                idxv[pl.ds(g * 16, 16)] >> jnp.full((16,), 4, jnp.int32))
        pltpu.sync_copy(valm_h.at[pl.ds(off, C)], vbuf)
        pltpu.sync_copy(vbuf, accm.at[idxv], add=True)
        pltpu.sync_copy(valt_h.at[pl.ds(off, C)], vbuf)
        pltpu.sync_copy(vbuf, acct.at[idxt], add=True)

    plsc.subcore_barrier()

    @pl.loop(0, _MZCH)
    def _woutm(k):
        pltpu.sync_copy(accm.at[pl.ds(mrow0 + k * C, C)],
                        outm_h.at[cid, pl.ds(mrow0 + k * C, C)])

    pltpu.sync_copy(acct.at[pl.ds(trow0, _TROWS)],
                    outt_h.at[cid, pl.ds(trow0, _TROWS)])


def _sc_scatter(vals_m, vals_t, dstp):
    return pl.kernel(
        _scatter_body,
        out_type=(
            jax.ShapeDtypeStruct((NC, NP, 128), _f32),
            jax.ShapeDtypeStruct((NC, NPG, 128), _f32),
        ),
        mesh=_sc_mesh(),
        scratch_types=[
            pltpu.VMEM((C,), jnp.int32),
            pltpu.VMEM((C,), jnp.int32),
            pltpu.VMEM((C, 128), _f32),
            pltpu.VMEM_SHARED((NP, 128), _f32),
            pltpu.VMEM_SHARED((NPG, 128), _f32),
        ],
    )(vals_m, vals_t, dstp)


def _iota_f32(cols):
    return lax.broadcasted_iota(jnp.int32, (NB, cols), 1).astype(_f32)


def _com_first_body(c_ref, nd_ref, com_ref, acc):
    i = pl.program_id(0)

    @pl.when(i == 0)
    def _():
        acc[...] = jnp.zeros_like(acc)

    nd = nd_ref[...]
    gidf = nd[:, 0:1]
    m = nd[:, 1:2]
    cc = c_ref[...][:, 0:3]
    oh = (gidf == _iota_f32(G)).astype(_f32)
    vals = jnp.concatenate([m * cc, m, jnp.zeros((NB, 4), _f32)], axis=1)
    acc[...] += lax.dot_general(oh, vals, (((0,), (0,)), ((), ())),
                                preferred_element_type=_f32)

    @pl.when(i == NBLK - 1)
    def _():
        a = acc[...]
        com = a[:, 0:3] / jnp.clip(a[:, 3:4], 1e-6, None)
        com_ref[...] = jnp.concatenate([com, jnp.zeros((G, 5), _f32)], axis=1)


def _com_first(coords8, nodes8):
    return pl.pallas_call(
        _com_first_body,
        grid=(NBLK,),
        in_specs=[
            pl.BlockSpec((NB, 8), lambda i: (i, 0)),
            pl.BlockSpec((NB, 8), lambda i: (i, 0)),
        ],
        out_specs=pl.BlockSpec((G, 8), lambda i: (0, 0)),
        out_shape=jax.ShapeDtypeStruct((G, 8), _f32),
        scratch_shapes=[pltpu.VMEM((G, 8), _f32)],
    )(coords8, nodes8)


def _com_next_body(cz_ref, atA_ref, atB_ref, nd_ref, com_ref, cnew_ref, acc):
    i = pl.program_id(0)

    @pl.when(i == 0)
    def _():
        acc[...] = jnp.zeros_like(acc)

    nd = nd_ref[...]
    gidf = nd[:, 0:1]
    m = nd[:, 1:2]
    cnew = (cz_ref[...][:, 0:3]
            + atA_ref[...][:, 0:3] + atB_ref[...][:, 0:3])
    cnew_ref[...] = jnp.concatenate([cnew, jnp.zeros((NB, 5), _f32)], axis=1)
    oh = (gidf == _iota_f32(G)).astype(_f32)
    vals = jnp.concatenate([m * cnew, m, jnp.zeros((NB, 4), _f32)], axis=1)
    acc[...] += lax.dot_general(oh, vals, (((0,), (0,)), ((), ())),
                                preferred_element_type=_f32)

    @pl.when(i == NBLK - 1)
    def _():
        a = acc[...]
        com = a[:, 0:3] / jnp.clip(a[:, 3:4], 1e-6, None)
        com_ref[...] = jnp.concatenate([com, jnp.zeros((G, 5), _f32)], axis=1)


def _com_next(cz8, aggtA, aggtB, nodes8):
    return pl.pallas_call(
        _com_next_body,
        grid=(NBLK,),
        in_specs=[
            pl.BlockSpec((NB, 8), lambda i: (i, 0)),
            pl.BlockSpec((NB, 8), lambda i: (i, 0)),
            pl.BlockSpec((NB, 8), lambda i: (i, 0)),
            pl.BlockSpec((NB, 8), lambda i: (i, 0)),
        ],
        out_specs=[
            pl.BlockSpec((G, 8), lambda i: (0, 0)),
            pl.BlockSpec((NB, 8), lambda i: (i, 0)),
        ],
        out_shape=[
            jax.ShapeDtypeStruct((G, 8), _f32),
            jax.ShapeDtypeStruct((NP, 8), _f32),
        ],
        scratch_shapes=[pltpu.VMEM((G, 8), _f32)],
    )(cz8, aggtA, aggtB, nodes8)


def _tables(h, cz, craw, Wd, Ws, eb1):
    zpad = jnp.zeros((NB, TW - 134), _f32)
    td = jnp.concatenate(
        [jnp.dot(h, Wd, preferred_element_type=_f32) + eb1, cz, craw, zpad],
        axis=1)
    ts = jnp.concatenate(
        [jnp.dot(h, Ws, preferred_element_type=_f32), -cz, -craw, zpad],
        axis=1)
    return td, ts


def _init_body(nd_ref, craw_ref, f16_ref, com_ref, embp_ref, pWt_ref, pWb_ref,
               pb_ref, Wd_ref, Ws_ref, eb1_ref, h_ref, td_ref, ts_ref, cz_ref):
    nd = nd_ref[...]
    gidf = nd[:, 0:1]
    anf = nd[:, 2:3]
    ohA = (anf == _iota_f32(128)).astype(_f32)
    tbl = jnp.dot(embp_ref[...], pWt_ref[...], preferred_element_type=_f32)
    h = (jnp.dot(ohA, tbl, preferred_element_type=_f32)
         + jnp.dot(f16_ref[...], pWb_ref[...], preferred_element_type=_f32)
         + pb_ref[...])
    ohG = (gidf == _iota_f32(G)).astype(_f32)
    craw = craw_ref[...][:, 0:3]
    cz = craw - jnp.dot(ohG, com_ref[...], preferred_element_type=_f32)[:, 0:3]
    h_ref[...] = h
    td_ref[...], ts_ref[...] = _tables(h, cz, craw, Wd_ref[...], Ws_ref[...],
                                       eb1_ref[...])
    cz_ref[...] = jnp.concatenate([cz, jnp.zeros((NB, 5), _f32)], axis=1)


def _full(shp):
    return pl.BlockSpec(shp, lambda i: tuple(0 for _ in shp))


def _init(nodes8, coords8, feat16, com0, embp, pWt, pWb, pb, Wd, Ws, eb1):
    return pl.pallas_call(
        _init_body,
        grid=(NBLK,),
        in_specs=[
            pl.BlockSpec((NB, 8), lambda i: (i, 0)),
            pl.BlockSpec((NB, 8), lambda i: (i, 0)),
            pl.BlockSpec((NB, 16), lambda i: (i, 0)),
            _full((G, 8)),
            _full((128, 128)),
            _full((128, 128)),
            _full((16, 128)),
            _full((1, 128)),
            _full((128, 128)),
            _full((128, 128)),
            _full((1, 128)),
        ],
        out_specs=[
            pl.BlockSpec((NB, 128), lambda i: (i, 0)),
            pl.BlockSpec((NB, TW), lambda i: (i, 0)),
            pl.BlockSpec((NB, TW), lambda i: (i, 0)),
            pl.BlockSpec((NB, 8), lambda i: (i, 0)),
        ],
        out_shape=[
            jax.ShapeDtypeStruct((NP, 128), _f32),
            jax.ShapeDtypeStruct((NP, TW), _f32),
            jax.ShapeDtypeStruct((NP, TW), _f32),
            jax.ShapeDtypeStruct((NP, 8), _f32),
        ],
    )(nodes8, coords8, feat16, com0, embp, pWt, pWb, pb, Wd, Ws, eb1)


def _edge_core(sd, a, dstv, wa, eW2, eb2, xW1, xb1, xW2p):
    hsum = sd[:, 0:128]
    diff = sd[:, 128:131]
    v = hsum + a * wa
    m1 = jax.nn.silu(v)
    m = jax.nn.silu(jnp.dot(m1, eW2, preferred_element_type=_f32) + eb2)
    t = jax.nn.silu(jnp.dot(m, xW1, preferred_element_type=_f32) + xb1)
    w = jnp.dot(t, xW2p, preferred_element_type=_f32)[:, 0:1]
    dist = jnp.sqrt(jnp.sum(diff * diff, axis=1, keepdims=True) + 1e-8)
    trans = diff / (dist + 1.0) * w
    off = (dstv & 15) * 8
    lanes = lax.broadcasted_iota(jnp.int32, (sd.shape[0], 128), 1)
    packed = (jnp.where(lanes == off, trans[:, 0:1], 0.0)
              + jnp.where(lanes == off + 1, trans[:, 1:2], 0.0)
              + jnp.where(lanes == off + 2, trans[:, 2:3], 0.0))
    return m, packed


def _edge_a1_body(gd_ref, gs_ref, dst_ref, wa_ref, eW2_ref, eb2_ref, xW1_ref,
                  xb1_ref, xW2_ref, vm_ref, vt_ref, a_ref):
    sd = gd_ref[...] + gs_ref[...]
    ar = sd[:, 131:134]
    a = jnp.sum(ar * ar, axis=1, keepdims=True)
    vm_ref[...], vt_ref[...] = _edge_core(
        sd, a, dst_ref[...], wa_ref[...], eW2_ref[...], eb2_ref[...],
        xW1_ref[...], xb1_ref[...], xW2_ref[...])
    a_ref[...] = a


def _edge_a2_body(gd_ref, gs_ref, dst_ref, a_in_ref, wa_ref, eW2_ref, eb2_ref,
                  xW1_ref, xb1_ref, xW2_ref, vm_ref, vt_ref):
    sd = gd_ref[...] + gs_ref[...]
    vm_ref[...], vt_ref[...] = _edge_core(
        sd, a_in_ref[...], dst_ref[...], wa_ref[...], eW2_ref[...],
        eb2_ref[...], xW1_ref[...], xb1_ref[...], xW2_ref[...])


def _edge_weight_specs():
    return [_full((1, 128)), _full((128, 128)), _full((1, 128)),
            _full((128, 128)), _full((1, 128)), _full((128, 8))]


_EDGE_OUT_MT = [
    pl.BlockSpec((TE, 128), lambda i: (i, 0)),
    pl.BlockSpec((TE, 128), lambda i: (i, 0)),
]


def _edge_a1(gd, gs, dst2, wa, eW2, eb2, xW1, xb1, xW2p):
    return pl.pallas_call(
        _edge_a1_body,
        grid=(NEB,),
        in_specs=[
            pl.BlockSpec((TE, TW), lambda i: (i, 0)),
            pl.BlockSpec((TE, TW), lambda i: (i, 0)),
            pl.BlockSpec((TE, 1), lambda i: (i, 0)),
        ] + _edge_weight_specs(),
        out_specs=_EDGE_OUT_MT + [pl.BlockSpec((TE, 1), lambda i: (i, 0))],
        out_shape=[
            jax.ShapeDtypeStruct((EP, 128), _f32),
            jax.ShapeDtypeStruct((EP, 128), _f32),
            jax.ShapeDtypeStruct((EP, 1), _f32),
        ],
    )(gd, gs, dst2, wa, eW2, eb2, xW1, xb1, xW2p)


def _edge_a2(gd, gs, dst2, a_edges, wa, eW2, eb2, xW1, xb1, xW2p):
    return pl.pallas_call(
        _edge_a2_body,
        grid=(NEB,),
        in_specs=[
            pl.BlockSpec((TE, TW), lambda i: (i, 0)),
            pl.BlockSpec((TE, TW), lambda i: (i, 0)),
            pl.BlockSpec((TE, 1), lambda i: (i, 0)),
            pl.BlockSpec((TE, 1), lambda i: (i, 0)),
        ] + _edge_weight_specs(),
        out_specs=_EDGE_OUT_MT,
        out_shape=[
            jax.ShapeDtypeStruct((EP, 128), _f32),
            jax.ShapeDtypeStruct((EP, 128), _f32),
        ],
    )(gd, gs, dst2, a_edges, wa, eW2, eb2, xW1, xb1, xW2p)


def _b2_body(h_ref, amA_ref, amB_ref, cnew_ref, nd_ref, com_ref, hW1a_ref,
             hW1b_ref, hb1_ref, hW2_ref, hb2_ref, Wd_ref, Ws_ref, eb1_ref,
             hn_ref, td_ref, ts_ref, cz_ref):
    h = h_ref[...]
    aggm = amA_ref[...] + amB_ref[...]
    u = jax.nn.silu(jnp.dot(h, hW1a_ref[...], preferred_element_type=_f32)
                    + jnp.dot(aggm, hW1b_ref[...], preferred_element_type=_f32)
                    + hb1_ref[...])
    hn = h + jnp.dot(u, hW2_ref[...], preferred_element_type=_f32) + hb2_ref[...]
    nd = nd_ref[...]
    gidf = nd[:, 0:1]
    ohG = (gidf == _iota_f32(G)).astype(_f32)
    cz = (cnew_ref[...][:, 0:3]
          - jnp.dot(ohG, com_ref[...], preferred_element_type=_f32)[:, 0:3])
    hn_ref[...] = hn
    zcraw = jnp.zeros((NB, 3), _f32)
    td_ref[...], ts_ref[...] = _tables(hn, cz, zcraw, Wd_ref[...], Ws_ref[...],
                                       eb1_ref[...])
    cz_ref[...] = jnp.concatenate([cz, jnp.zeros((NB, 5), _f32)], axis=1)


def _b2(h, amA, amB, cnew8, nodes8, com, hW1a, hW1b, hb1, hW2, hb2, Wd, Ws,
        eb1):
    return pl.pallas_call(
        _b2_body,
        grid=(NBLK,),
        in_specs=[
            pl.BlockSpec((NB, 128), lambda i: (i, 0)),
            pl.BlockSpec((NB, 128), lambda i: (i, 0)),
            pl.BlockSpec((NB, 128), lambda i: (i, 0)),
            pl.BlockSpec((NB, 8), lambda i: (i, 0)),
            pl.BlockSpec((NB, 8), lambda i: (i, 0)),
            _full((G, 8)),
            _full((128, 128)), _full((128, 128)), _full((1, 128)),
            _full((128, 128)), _full((1, 128)),
            _full((128, 128)), _full((128, 128)), _full((1, 128)),
        ],
        out_specs=[
            pl.BlockSpec((NB, 128), lambda i: (i, 0)),
            pl.BlockSpec((NB, TW), lambda i: (i, 0)),
            pl.BlockSpec((NB, TW), lambda i: (i, 0)),
            pl.BlockSpec((NB, 8), lambda i: (i, 0)),
        ],
        out_shape=[
            jax.ShapeDtypeStruct((NP, 128), _f32),
            jax.ShapeDtypeStruct((NP, TW), _f32),
            jax.ShapeDtypeStruct((NP, TW), _f32),
            jax.ShapeDtypeStruct((NP, 8), _f32),
        ],
    )(h, amA, amB, cnew8, nodes8, com, hW1a, hW1b, hb1, hW2, hb2, Wd, Ws, eb1)


def _final_body(cnew_ref, nd_ref, com_ref, out_ref):
    nd = nd_ref[...]
    gidf = nd[:, 0:1]
    ohG = (gidf == _iota_f32(G)).astype(_f32)
    cz = (cnew_ref[...][:, 0:3]
          - jnp.dot(ohG, com_ref[...], preferred_element_type=_f32)[:, 0:3])
    out_ref[...] = jnp.concatenate([cz, jnp.zeros((NB, 5), _f32)], axis=1)


def _final(cnew8, nodes8, com):
    return pl.pallas_call(
        _final_body,
        grid=(NBLK,),
        in_specs=[
            pl.BlockSpec((NB, 8), lambda i: (i, 0)),
            pl.BlockSpec((NB, 8), lambda i: (i, 0)),
            _full((G, 8)),
        ],
        out_specs=pl.BlockSpec((NB, 8), lambda i: (i, 0)),
        out_shape=jax.ShapeDtypeStruct((NP, 8), _f32),
    )(cnew8, nodes8, com)


def _pad_edges(idx):
    per = idx.reshape(NW, E // NW)
    per = jnp.pad(per, ((0, 0), (0, EWP - E // NW)), constant_values=N)
    return per.reshape(EP).astype(jnp.int32)


def kernel(temb, masses, masses_normalized, cond_labels, cond_mask, moments,
           coords, emb, proj_W, proj_b, eW1, eb1, eW2, eb2, xW1, xb1, xW2,
           hW1, hb1, hW2, hb2, atom_nums, edge_index, node_graph_idx):
    srcp = _pad_edges(edge_index[0])
    dstp = _pad_edges(edge_index[1])
    dst2 = dstp[:, None]

    gidf = node_graph_idx.astype(_f32)[:, None]
    anf = atom_nums.astype(_f32)[:, None]
    nodes8 = jnp.pad(
        jnp.concatenate([gidf, masses, anf], axis=1), ((0, NP - N), (0, 5)))
    coords8 = jnp.pad(coords, ((0, NP - N), (0, 5)))
    feat16 = jnp.pad(
        jnp.concatenate([temb, masses, masses_normalized, cond_labels,
                         cond_mask, moments], axis=1), ((0, NP - N), (0, 4)))
    embp = jnp.pad(emb, ((0, 128 - emb.shape[0]), (0, 0)))
    pWt = proj_W[:AF]
    pWb = jnp.pad(proj_W[AF:], ((0, 4), (0, 0)))
    pb = proj_b[None, :]
    xW2p = jnp.pad(xW2, ((0, 0), (0, 0), (0, 7)))

    com0 = _com_first(coords8, nodes8)
    h, td, ts, cz8 = _init(nodes8, coords8, feat16, com0, embp, pWt, pWb, pb,
                           eW1[0, 0:H], eW1[0, H:2 * H], eb1[0][None, :])

    a_edges = None
    out = None
    for i in range(LAYERS):
        gd, gs = _sc_gather(td, ts, dstp, srcp)
        wa = eW1[i, 2 * H:2 * H + 1]
        if i == 0:
            vals_m, vals_t, a_edges = _edge_a1(
                gd, gs, dst2, wa, eW2[i], eb2[i][None, :], xW1[i],
                xb1[i][None, :], xW2p[i])
        else:
            vals_m, vals_t = _edge_a2(
                gd, gs, dst2, a_edges, wa, eW2[i], eb2[i][None, :], xW1[i],
                xb1[i][None, :], xW2p[i])
        out_m, out_t = _sc_scatter(vals_m, vals_t, dstp)
        aggtA = out_t[0].reshape(NP, 8)
        aggtB = out_t[1].reshape(NP, 8)
        com, cnew8 = _com_next(cz8, aggtA, aggtB, nodes8)
        if i + 1 < LAYERS:
            h, td, ts, cz8 = _b2(h, out_m[0], out_m[1], cnew8, nodes8, com,
                                 hW1[i, 0:H], hW1[i, H:2 * H],
                                 hb1[i][None, :], hW2[i], hb2[i][None, :],
                                 eW1[i + 1, 0:H], eW1[i + 1, H:2 * H],
                                 eb1[i + 1][None, :])
        else:
            out = _final(cnew8, nodes8, com)
    return out[:N, 0:3]


# speedup vs baseline: 2.4943x; 1.0251x over previous
import jax
import jax.numpy as jnp
from jax import lax
from jax.experimental import pallas as pl
from jax.experimental.pallas import tpu as pltpu
from jax.experimental.pallas import tpu_sc as plsc

N = 10000
E = 320000
G = 256
H = 128
AF = 128
LAYERS = 4

NC = 2
NS = 16
NW = NC * NS
EWP = 10240
EP = NW * EWP
CG = 64
NCHG = EWP // CG
CS = 64
NCHS = EWP // CS
NP = 10240
NPG = NP // 16
TW = 256
NB = 512
NBLK = NP // NB
TE = 1024
NEB = EP // TE

_f32 = jnp.float32


def _sc_mesh():
    return plsc.VectorSubcoreMesh(
        core_axis_name="c", subcore_axis_name="s", num_cores=NC, num_subcores=NS
    )


def _gather_body(td_h, ts_h, dst_h, src_h, od_h, os_h,
                 idxd0, idxd1, idxs0, idxs1, rowd0, rowd1, rows0, rows1,
                 sgd0, sgd1, sgs0, sgs1, sod0, sod1, sos0, sos1):
    wid = lax.axis_index("s") * NC + lax.axis_index("c")
    base = wid * EWP
    idxd = (idxd0, idxd1)
    idxs = (idxs0, idxs1)
    rowd = (rowd0, rowd1)
    rows = (rows0, rows1)
    sgd = (sgd0, sgd1)
    sgs = (sgs0, sgs1)
    sod = (sod0, sod1)
    sos = (sos0, sos1)

    def start(chunk, b):
        off = base + chunk * CG
        pltpu.sync_copy(dst_h.at[pl.ds(off, CG)], idxd[b])
        pltpu.async_copy(td_h.at[idxd[b]], rowd[b], sgd[b])
        pltpu.sync_copy(src_h.at[pl.ds(off, CG)], idxs[b])
        pltpu.async_copy(ts_h.at[idxs[b]], rows[b], sgs[b])

    def wait_gather(b):
        pltpu.make_async_copy(td_h.at[idxd[b]], rowd[b], sgd[b]).wait()
        pltpu.make_async_copy(ts_h.at[idxs[b]], rows[b], sgs[b]).wait()

    def start_out(chunk, b):
        off = base + chunk * CG
        pltpu.async_copy(rowd[b], od_h.at[pl.ds(off, CG)], sod[b])
        pltpu.async_copy(rows[b], os_h.at[pl.ds(off, CG)], sos[b])

    def wait_out(b):
        pltpu.make_async_copy(rowd[b], od_h.at[pl.ds(base, CG)], sod[b]).wait()
        pltpu.make_async_copy(rows[b], os_h.at[pl.ds(base, CG)], sos[b]).wait()

    start(0, 0)

    @pl.loop(0, NCHG, step=2)
    def _pair(j):
        for b in range(2):
            cur = j + b
            nxt = cur + 1

            @pl.when(nxt < NCHG)
            def _():
                @pl.when(nxt >= 2)
                def _():
                    wait_out(1 - b)
                start(nxt, 1 - b)

            wait_gather(b)
            start_out(cur, b)

    wait_out(0)
    wait_out(1)


def _sc_gather(td, ts, dstp, srcp):
    return pl.kernel(
        _gather_body,
        out_type=(
            jax.ShapeDtypeStruct((EP, TW), _f32),
            jax.ShapeDtypeStruct((EP, TW), _f32),
        ),
        mesh=_sc_mesh(),
        scratch_types=[
            pltpu.VMEM((CG,), jnp.int32),
            pltpu.VMEM((CG,), jnp.int32),
            pltpu.VMEM((CG,), jnp.int32),
            pltpu.VMEM((CG,), jnp.int32),
            pltpu.VMEM((CG, TW), _f32),
            pltpu.VMEM((CG, TW), _f32),
            pltpu.VMEM((CG, TW), _f32),
            pltpu.VMEM((CG, TW), _f32),
        ] + [pltpu.SemaphoreType.DMA] * 8,
    )(td, ts, dstp, srcp)


_MROWS = NP // NS
_MZCH = _MROWS // CS
_TROWS = NPG // NS


def _scatter_body(valm_h, valt_h, dst_h, outm_h, outt_h,
                  idx0, idx1, idxt0, idxt1, vm0, vm1, vt0, vt1,
                  accm, acct, sl0, sl1):
    cid = lax.axis_index("c")
    sid = lax.axis_index("s")
    wid = sid * NC + cid
    base = wid * EWP
    mrow0 = sid * _MROWS
    trow0 = sid * _TROWS
    idx = (idx0, idx1)
    idxt = (idxt0, idxt1)
    vm = (vm0, vm1)
    vt = (vt0, vt1)
    sl = (sl0, sl1)

    zero16 = jnp.zeros((16,), _f32)

    @pl.loop(0, CS)
    def _zrow(i):
        for c in range(128 // 16):
            vm0[i, pl.ds(c * 16, 16)] = zero16

    @pl.loop(0, _MZCH)
    def _zaccm(k):
        pltpu.sync_copy(vm0, accm.at[pl.ds(mrow0 + k * CS, CS)])

    pltpu.sync_copy(vm0.at[pl.ds(0, _TROWS)], acct.at[pl.ds(trow0, _TROWS)])

    plsc.subcore_barrier()

    def start_loads(chunk, b):
        off = base + chunk * CS
        pltpu.async_copy(dst_h.at[pl.ds(off, CS)], idx[b], sl[b])
        pltpu.async_copy(valm_h.at[pl.ds(off, CS)], vm[b], sl[b])
        pltpu.async_copy(valt_h.at[pl.ds(off, CS)], vt[b], sl[b])

    def wait_loads(b):
        pltpu.make_async_copy(dst_h.at[pl.ds(base, CS)], idx[b], sl[b]).wait()
        pltpu.make_async_copy(valm_h.at[pl.ds(base, CS)], vm[b], sl[b]).wait()
        pltpu.make_async_copy(valt_h.at[pl.ds(base, CS)], vt[b], sl[b]).wait()

    start_loads(0, 0)

    @pl.loop(0, NCHS, step=2)
    def _pair(j):
        for b in range(2):
            cur = j + b
            nxt = cur + 1

            @pl.when(nxt < NCHS)
            def _():
                start_loads(nxt, 1 - b)

            wait_loads(b)
            for g in range(CS // 16):
                idxt[b][pl.ds(g * 16, 16)] = (
                    idx[b][pl.ds(g * 16, 16)] >> jnp.full((16,), 4, jnp.int32))
            pltpu.sync_copy(vm[b], accm.at[idx[b]], add=True)
            pltpu.sync_copy(vt[b], acct.at[idxt[b]], add=True)

    plsc.subcore_barrier()

    @pl.loop(0, _MZCH)
    def _woutm(k):
        pltpu.sync_copy(accm.at[pl.ds(mrow0 + k * CS, CS)],
                        outm_h.at[cid, pl.ds(mrow0 + k * CS, CS)])

    pltpu.sync_copy(acct.at[pl.ds(trow0, _TROWS)],
                    outt_h.at[cid, pl.ds(trow0, _TROWS)])


def _sc_scatter(vals_m, vals_t, dstp):
    return pl.kernel(
        _scatter_body,
        out_type=(
            jax.ShapeDtypeStruct((NC, NP, 128), _f32),
            jax.ShapeDtypeStruct((NC, NPG, 128), _f32),
        ),
        mesh=_sc_mesh(),
        scratch_types=[
            pltpu.VMEM((CS,), jnp.int32),
            pltpu.VMEM((CS,), jnp.int32),
            pltpu.VMEM((CS,), jnp.int32),
            pltpu.VMEM((CS,), jnp.int32),
            pltpu.VMEM((CS, 128), _f32),
            pltpu.VMEM((CS, 128), _f32),
            pltpu.VMEM((CS, 128), _f32),
            pltpu.VMEM((CS, 128), _f32),
            pltpu.VMEM_SHARED((NP, 128), _f32),
            pltpu.VMEM_SHARED((NPG, 128), _f32),
            pltpu.SemaphoreType.DMA,
            pltpu.SemaphoreType.DMA,
        ],
    )(vals_m, vals_t, dstp)


def _iota_f32(cols):
    return lax.broadcasted_iota(jnp.int32, (NB, cols), 1).astype(_f32)


def _com_first_body(c_ref, nd_ref, com_ref, acc):
    i = pl.program_id(0)

    @pl.when(i == 0)
    def _():
        acc[...] = jnp.zeros_like(acc)

    nd = nd_ref[...]
    gidf = nd[:, 0:1]
    m = nd[:, 1:2]
    cc = c_ref[...][:, 0:3]
    oh = (gidf == _iota_f32(G)).astype(_f32)
    vals = jnp.concatenate([m * cc, m, jnp.zeros((NB, 4), _f32)], axis=1)
    acc[...] += lax.dot_general(oh, vals, (((0,), (0,)), ((), ())),
                                preferred_element_type=_f32)

    @pl.when(i == NBLK - 1)
    def _():
        a = acc[...]
        com = a[:, 0:3] / jnp.clip(a[:, 3:4], 1e-6, None)
        com_ref[...] = jnp.concatenate([com, jnp.zeros((G, 5), _f32)], axis=1)


def _com_first(coords8, nodes8):
    return pl.pallas_call(
        _com_first_body,
        grid=(NBLK,),
        in_specs=[
            pl.BlockSpec((NB, 8), lambda i: (i, 0)),
            pl.BlockSpec((NB, 8), lambda i: (i, 0)),
        ],
        out_specs=pl.BlockSpec((G, 8), lambda i: (0, 0)),
        out_shape=jax.ShapeDtypeStruct((G, 8), _f32),
        scratch_shapes=[pltpu.VMEM((G, 8), _f32)],
    )(coords8, nodes8)


def _com_next_body(cz_ref, atA_ref, atB_ref, nd_ref, com_ref, cnew_ref, acc):
    i = pl.program_id(0)

    @pl.when(i == 0)
    def _():
        acc[...] = jnp.zeros_like(acc)

    nd = nd_ref[...]
    gidf = nd[:, 0:1]
    m = nd[:, 1:2]
    cnew = (cz_ref[...][:, 0:3]
            + atA_ref[...][:, 0:3] + atB_ref[...][:, 0:3])
    cnew_ref[...] = jnp.concatenate([cnew, jnp.zeros((NB, 5), _f32)], axis=1)
    oh = (gidf == _iota_f32(G)).astype(_f32)
    vals = jnp.concatenate([m * cnew, m, jnp.zeros((NB, 4), _f32)], axis=1)
    acc[...] += lax.dot_general(oh, vals, (((0,), (0,)), ((), ())),
                                preferred_element_type=_f32)

    @pl.when(i == NBLK - 1)
    def _():
        a = acc[...]
        com = a[:, 0:3] / jnp.clip(a[:, 3:4], 1e-6, None)
        com_ref[...] = jnp.concatenate([com, jnp.zeros((G, 5), _f32)], axis=1)


def _com_next(cz8, aggtA, aggtB, nodes8):
    return pl.pallas_call(
        _com_next_body,
        grid=(NBLK,),
        in_specs=[
            pl.BlockSpec((NB, 8), lambda i: (i, 0)),
            pl.BlockSpec((NB, 8), lambda i: (i, 0)),
            pl.BlockSpec((NB, 8), lambda i: (i, 0)),
            pl.BlockSpec((NB, 8), lambda i: (i, 0)),
        ],
        out_specs=[
            pl.BlockSpec((G, 8), lambda i: (0, 0)),
            pl.BlockSpec((NB, 8), lambda i: (i, 0)),
        ],
        out_shape=[
            jax.ShapeDtypeStruct((G, 8), _f32),
            jax.ShapeDtypeStruct((NP, 8), _f32),
        ],
        scratch_shapes=[pltpu.VMEM((G, 8), _f32)],
    )(cz8, aggtA, aggtB, nodes8)


def _tables(h, cz, craw, Wd, Ws, eb1):
    zpad = jnp.zeros((NB, TW - 134), _f32)
    td = jnp.concatenate(
        [jnp.dot(h, Wd, preferred_element_type=_f32) + eb1, cz, craw, zpad],
        axis=1)
    ts = jnp.concatenate(
        [jnp.dot(h, Ws, preferred_element_type=_f32), -cz, -craw, zpad],
        axis=1)
    return td, ts


def _init_body(nd_ref, craw_ref, f16_ref, com_ref, embp_ref, pWt_ref, pWb_ref,
               pb_ref, Wd_ref, Ws_ref, eb1_ref, h_ref, td_ref, ts_ref, cz_ref):
    nd = nd_ref[...]
    gidf = nd[:, 0:1]
    anf = nd[:, 2:3]
    ohA = (anf == _iota_f32(128)).astype(_f32)
    tbl = jnp.dot(embp_ref[...], pWt_ref[...], preferred_element_type=_f32)
    h = (jnp.dot(ohA, tbl, preferred_element_type=_f32)
         + jnp.dot(f16_ref[...], pWb_ref[...], preferred_element_type=_f32)
         + pb_ref[...])
    ohG = (gidf == _iota_f32(G)).astype(_f32)
    craw = craw_ref[...][:, 0:3]
    cz = craw - jnp.dot(ohG, com_ref[...], preferred_element_type=_f32)[:, 0:3]
    h_ref[...] = h
    td_ref[...], ts_ref[...] = _tables(h, cz, craw, Wd_ref[...], Ws_ref[...],
                                       eb1_ref[...])
    cz_ref[...] = jnp.concatenate([cz, jnp.zeros((NB, 5), _f32)], axis=1)


def _full(shp):
    return pl.BlockSpec(shp, lambda i: tuple(0 for _ in shp))


def _init(nodes8, coords8, feat16, com0, embp, pWt, pWb, pb, Wd, Ws, eb1):
    return pl.pallas_call(
        _init_body,
        grid=(NBLK,),
        in_specs=[
            pl.BlockSpec((NB, 8), lambda i: (i, 0)),
            pl.BlockSpec((NB, 8), lambda i: (i, 0)),
            pl.BlockSpec((NB, 16), lambda i: (i, 0)),
            _full((G, 8)),
            _full((128, 128)),
            _full((128, 128)),
            _full((16, 128)),
            _full((1, 128)),
            _full((128, 128)),
            _full((128, 128)),
            _full((1, 128)),
        ],
        out_specs=[
            pl.BlockSpec((NB, 128), lambda i: (i, 0)),
            pl.BlockSpec((NB, TW), lambda i: (i, 0)),
            pl.BlockSpec((NB, TW), lambda i: (i, 0)),
            pl.BlockSpec((NB, 8), lambda i: (i, 0)),
        ],
        out_shape=[
            jax.ShapeDtypeStruct((NP, 128), _f32),
            jax.ShapeDtypeStruct((NP, TW), _f32),
            jax.ShapeDtypeStruct((NP, TW), _f32),
            jax.ShapeDtypeStruct((NP, 8), _f32),
        ],
    )(nodes8, coords8, feat16, com0, embp, pWt, pWb, pb, Wd, Ws, eb1)


def _edge_core(sd, a, dstv, wa, eW2, eb2, xW1, xb1, xW2p):
    hsum = sd[:, 0:128]
    diff = sd[:, 128:131]
    v = hsum + a * wa
    m1 = jax.nn.silu(v)
    m = jax.nn.silu(jnp.dot(m1, eW2, preferred_element_type=_f32) + eb2)
    t = jax.nn.silu(jnp.dot(m, xW1, preferred_element_type=_f32) + xb1)
    w = jnp.dot(t, xW2p, preferred_element_type=_f32)[:, 0:1]
    dist = jnp.sqrt(jnp.sum(diff * diff, axis=1, keepdims=True) + 1e-8)
    trans = diff / (dist + 1.0) * w
    off = (dstv & 15) * 8
    lanes = lax.broadcasted_iota(jnp.int32, (sd.shape[0], 128), 1)
    packed = (jnp.where(lanes == off, trans[:, 0:1], 0.0)
              + jnp.where(lanes == off + 1, trans[:, 1:2], 0.0)
              + jnp.where(lanes == off + 2, trans[:, 2:3], 0.0))
    return m, packed


def _edge_a1_body(gd_ref, gs_ref, dst_ref, wa_ref, eW2_ref, eb2_ref, xW1_ref,
                  xb1_ref, xW2_ref, vm_ref, vt_ref, a_ref):
    sd = gd_ref[...] + gs_ref[...]
    ar = sd[:, 131:134]
    a = jnp.sum(ar * ar, axis=1, keepdims=True)
    vm_ref[...], vt_ref[...] = _edge_core(
        sd, a, dst_ref[...], wa_ref[...], eW2_ref[...], eb2_ref[...],
        xW1_ref[...], xb1_ref[...], xW2_ref[...])
    a_ref[...] = a


def _edge_a2_body(gd_ref, gs_ref, dst_ref, a_in_ref, wa_ref, eW2_ref, eb2_ref,
                  xW1_ref, xb1_ref, xW2_ref, vm_ref, vt_ref):
    sd = gd_ref[...] + gs_ref[...]
    vm_ref[...], vt_ref[...] = _edge_core(
        sd, a_in_ref[...], dst_ref[...], wa_ref[...], eW2_ref[...],
        eb2_ref[...], xW1_ref[...], xb1_ref[...], xW2_ref[...])


def _edge_weight_specs():
    return [_full((1, 128)), _full((128, 128)), _full((1, 128)),
            _full((128, 128)), _full((1, 128)), _full((128, 8))]


_EDGE_OUT_MT = [
    pl.BlockSpec((TE, 128), lambda i: (i, 0)),
    pl.BlockSpec((TE, 128), lambda i: (i, 0)),
]


def _edge_a1(gd, gs, dst2, wa, eW2, eb2, xW1, xb1, xW2p):
    return pl.pallas_call(
        _edge_a1_body,
        grid=(NEB,),
        in_specs=[
            pl.BlockSpec((TE, TW), lambda i: (i, 0)),
            pl.BlockSpec((TE, TW), lambda i: (i, 0)),
            pl.BlockSpec((TE, 1), lambda i: (i, 0)),
        ] + _edge_weight_specs(),
        out_specs=_EDGE_OUT_MT + [pl.BlockSpec((TE, 1), lambda i: (i, 0))],
        out_shape=[
            jax.ShapeDtypeStruct((EP, 128), _f32),
            jax.ShapeDtypeStruct((EP, 128), _f32),
            jax.ShapeDtypeStruct((EP, 1), _f32),
        ],
    )(gd, gs, dst2, wa, eW2, eb2, xW1, xb1, xW2p)


def _edge_a2(gd, gs, dst2, a_edges, wa, eW2, eb2, xW1, xb1, xW2p):
    return pl.pallas_call(
        _edge_a2_body,
        grid=(NEB,),
        in_specs=[
            pl.BlockSpec((TE, TW), lambda i: (i, 0)),
            pl.BlockSpec((TE, TW), lambda i: (i, 0)),
            pl.BlockSpec((TE, 1), lambda i: (i, 0)),
            pl.BlockSpec((TE, 1), lambda i: (i, 0)),
        ] + _edge_weight_specs(),
        out_specs=_EDGE_OUT_MT,
        out_shape=[
            jax.ShapeDtypeStruct((EP, 128), _f32),
            jax.ShapeDtypeStruct((EP, 128), _f32),
        ],
    )(gd, gs, dst2, a_edges, wa, eW2, eb2, xW1, xb1, xW2p)


def _b2_body(h_ref, amA_ref, amB_ref, cnew_ref, nd_ref, com_ref, hW1a_ref,
             hW1b_ref, hb1_ref, hW2_ref, hb2_ref, Wd_ref, Ws_ref, eb1_ref,
             hn_ref, td_ref, ts_ref, cz_ref):
    h = h_ref[...]
    aggm = amA_ref[...] + amB_ref[...]
    u = jax.nn.silu(jnp.dot(h, hW1a_ref[...], preferred_element_type=_f32)
                    + jnp.dot(aggm, hW1b_ref[...], preferred_element_type=_f32)
                    + hb1_ref[...])
    hn = h + jnp.dot(u, hW2_ref[...], preferred_element_type=_f32) + hb2_ref[...]
    nd = nd_ref[...]
    gidf = nd[:, 0:1]
    ohG = (gidf == _iota_f32(G)).astype(_f32)
    cz = (cnew_ref[...][:, 0:3]
          - jnp.dot(ohG, com_ref[...], preferred_element_type=_f32)[:, 0:3])
    hn_ref[...] = hn
    zcraw = jnp.zeros((NB, 3), _f32)
    td_ref[...], ts_ref[...] = _tables(hn, cz, zcraw, Wd_ref[...], Ws_ref[...],
                                       eb1_ref[...])
    cz_ref[...] = jnp.concatenate([cz, jnp.zeros((NB, 5), _f32)], axis=1)


def _b2(h, amA, amB, cnew8, nodes8, com, hW1a, hW1b, hb1, hW2, hb2, Wd, Ws,
        eb1):
    return pl.pallas_call(
        _b2_body,
        grid=(NBLK,),
        in_specs=[
            pl.BlockSpec((NB, 128), lambda i: (i, 0)),
            pl.BlockSpec((NB, 128), lambda i: (i, 0)),
            pl.BlockSpec((NB, 128), lambda i: (i, 0)),
            pl.BlockSpec((NB, 8), lambda i: (i, 0)),
            pl.BlockSpec((NB, 8), lambda i: (i, 0)),
            _full((G, 8)),
            _full((128, 128)), _full((128, 128)), _full((1, 128)),
            _full((128, 128)), _full((1, 128)),
            _full((128, 128)), _full((128, 128)), _full((1, 128)),
        ],
        out_specs=[
            pl.BlockSpec((NB, 128), lambda i: (i, 0)),
            pl.BlockSpec((NB, TW), lambda i: (i, 0)),
            pl.BlockSpec((NB, TW), lambda i: (i, 0)),
            pl.BlockSpec((NB, 8), lambda i: (i, 0)),
        ],
        out_shape=[
            jax.ShapeDtypeStruct((NP, 128), _f32),
            jax.ShapeDtypeStruct((NP, TW), _f32),
            jax.ShapeDtypeStruct((NP, TW), _f32),
            jax.ShapeDtypeStruct((NP, 8), _f32),
        ],
    )(h, amA, amB, cnew8, nodes8, com, hW1a, hW1b, hb1, hW2, hb2, Wd, Ws, eb1)


def _final_body(cnew_ref, nd_ref, com_ref, out_ref):
    nd = nd_ref[...]
    gidf = nd[:, 0:1]
    ohG = (gidf == _iota_f32(G)).astype(_f32)
    cz = (cnew_ref[...][:, 0:3]
          - jnp.dot(ohG, com_ref[...], preferred_element_type=_f32)[:, 0:3])
    out_ref[...] = jnp.concatenate([cz, jnp.zeros((NB, 5), _f32)], axis=1)


def _final(cnew8, nodes8, com):
    return pl.pallas_call(
        _final_body,
        grid=(NBLK,),
        in_specs=[
            pl.BlockSpec((NB, 8), lambda i: (i, 0)),
            pl.BlockSpec((NB, 8), lambda i: (i, 0)),
            _full((G, 8)),
        ],
        out_specs=pl.BlockSpec((NB, 8), lambda i: (i, 0)),
        out_shape=jax.ShapeDtypeStruct((NP, 8), _f32),
    )(cnew8, nodes8, com)


def _pad_edges(idx):
    per = idx.reshape(NW, E // NW)
    per = jnp.pad(per, ((0, 0), (0, EWP - E // NW)), constant_values=N)
    return per.reshape(EP).astype(jnp.int32)


def kernel(temb, masses, masses_normalized, cond_labels, cond_mask, moments,
           coords, emb, proj_W, proj_b, eW1, eb1, eW2, eb2, xW1, xb1, xW2,
           hW1, hb1, hW2, hb2, atom_nums, edge_index, node_graph_idx):
    srcp = _pad_edges(edge_index[0])
    dstp = _pad_edges(edge_index[1])
    dst2 = dstp[:, None]

    gidf = node_graph_idx.astype(_f32)[:, None]
    anf = atom_nums.astype(_f32)[:, None]
    nodes8 = jnp.pad(
        jnp.concatenate([gidf, masses, anf], axis=1), ((0, NP - N), (0, 5)))
    coords8 = jnp.pad(coords, ((0, NP - N), (0, 5)))
    feat16 = jnp.pad(
        jnp.concatenate([temb, masses, masses_normalized, cond_labels,
                         cond_mask, moments], axis=1), ((0, NP - N), (0, 4)))
    embp = jnp.pad(emb, ((0, 128 - emb.shape[0]), (0, 0)))
    pWt = proj_W[:AF]
    pWb = jnp.pad(proj_W[AF:], ((0, 4), (0, 0)))
    pb = proj_b[None, :]
    xW2p = jnp.pad(xW2, ((0, 0), (0, 0), (0, 7)))

    com0 = _com_first(coords8, nodes8)
    h, td, ts, cz8 = _init(nodes8, coords8, feat16, com0, embp, pWt, pWb, pb,
                           eW1[0, 0:H], eW1[0, H:2 * H], eb1[0][None, :])

    a_edges = None
    out = None
    for i in range(LAYERS):
        gd, gs = _sc_gather(td, ts, dstp, srcp)
        wa = eW1[i, 2 * H:2 * H + 1]
        if i == 0:
            vals_m, vals_t, a_edges = _edge_a1(
                gd, gs, dst2, wa, eW2[i], eb2[i][None, :], xW1[i],
                xb1[i][None, :], xW2p[i])
        else:
            vals_m, vals_t = _edge_a2(
                gd, gs, dst2, a_edges, wa, eW2[i], eb2[i][None, :], xW1[i],
                xb1[i][None, :], xW2p[i])
        out_m, out_t = _sc_scatter(vals_m, vals_t, dstp)
        aggtA = out_t[0].reshape(NP, 8)
        aggtB = out_t[1].reshape(NP, 8)
        com, cnew8 = _com_next(cz8, aggtA, aggtB, nodes8)
        if i + 1 < LAYERS:
            h, td, ts, cz8 = _b2(h, out_m[0], out_m[1], cnew8, nodes8, com,
                                 hW1[i, 0:H], hW1[i, H:2 * H],
                                 hb1[i][None, :], hW2[i], hb2[i][None, :],
                                 eW1[i + 1, 0:H], eW1[i + 1, H:2 * H],
                                 eb1[i + 1][None, :])
        else:
            out = _final(cnew8, nodes8, com)
    return out[:N, 0:3]


# speedup vs baseline: 2.8525x; 1.1436x over previous
import jax
import jax.numpy as jnp
from jax import lax
from jax.experimental import pallas as pl
from jax.experimental.pallas import tpu as pltpu
from jax.experimental.pallas import tpu_sc as plsc

N = 10000
E = 320000
G = 256
H = 128
AF = 128
LAYERS = 4

NC = 2
NS = 16
NW = NC * NS
EWP = 10240
EP = NW * EWP
CG = 64
NCHG = EWP // CG
CS = 64
NCHS = EWP // CS
NP = 10240
NPG = NP // 16
TW = 256
NB = 512
NBLK = NP // NB
TE = 1024
NEB = EP // TE

_f32 = jnp.float32
_bf16 = jnp.bfloat16


def _sc_mesh():
    return plsc.VectorSubcoreMesh(
        core_axis_name="c", subcore_axis_name="s", num_cores=NC, num_subcores=NS
    )


def _gather_body(td_h, ts_h, dst_h, src_h, od_h, os_h,
                 idxd0, idxd1, idxs0, idxs1, rowd0, rowd1, rows0, rows1,
                 sgd0, sgd1, sgs0, sgs1, sod0, sod1, sos0, sos1):
    wid = lax.axis_index("s") * NC + lax.axis_index("c")
    base = wid * EWP
    idxd = (idxd0, idxd1)
    idxs = (idxs0, idxs1)
    rowd = (rowd0, rowd1)
    rows = (rows0, rows1)
    sgd = (sgd0, sgd1)
    sgs = (sgs0, sgs1)
    sod = (sod0, sod1)
    sos = (sos0, sos1)

    def start(chunk, b):
        off = base + chunk * CG
        pltpu.sync_copy(dst_h.at[pl.ds(off, CG)], idxd[b])
        pltpu.async_copy(td_h.at[idxd[b]], rowd[b], sgd[b])
        pltpu.sync_copy(src_h.at[pl.ds(off, CG)], idxs[b])
        pltpu.async_copy(ts_h.at[idxs[b]], rows[b], sgs[b])

    def wait_gather(b):
        pltpu.make_async_copy(td_h.at[idxd[b]], rowd[b], sgd[b]).wait()
        pltpu.make_async_copy(ts_h.at[idxs[b]], rows[b], sgs[b]).wait()

    def start_out(chunk, b):
        off = base + chunk * CG
        pltpu.async_copy(rowd[b], od_h.at[pl.ds(off, CG)], sod[b])
        pltpu.async_copy(rows[b], os_h.at[pl.ds(off, CG)], sos[b])

    def wait_out(b):
        pltpu.make_async_copy(rowd[b], od_h.at[pl.ds(base, CG)], sod[b]).wait()
        pltpu.make_async_copy(rows[b], os_h.at[pl.ds(base, CG)], sos[b]).wait()

    start(0, 0)

    @pl.loop(0, NCHG, step=2)
    def _pair(j):
        for b in range(2):
            cur = j + b
            nxt = cur + 1

            @pl.when(nxt < NCHG)
            def _():
                @pl.when(nxt >= 2)
                def _():
                    wait_out(1 - b)
                start(nxt, 1 - b)

            wait_gather(b)
            start_out(cur, b)

    wait_out(0)
    wait_out(1)


def _sc_gather(td, ts, dstp, srcp):
    return pl.kernel(
        _gather_body,
        out_type=(
            jax.ShapeDtypeStruct((EP, 128), jnp.int32),
            jax.ShapeDtypeStruct((EP, 128), jnp.int32),
        ),
        mesh=_sc_mesh(),
        scratch_types=[
            pltpu.VMEM((CG,), jnp.int32),
            pltpu.VMEM((CG,), jnp.int32),
            pltpu.VMEM((CG,), jnp.int32),
            pltpu.VMEM((CG,), jnp.int32),
            pltpu.VMEM((CG, 128), jnp.int32),
            pltpu.VMEM((CG, 128), jnp.int32),
            pltpu.VMEM((CG, 128), jnp.int32),
            pltpu.VMEM((CG, 128), jnp.int32),
        ] + [pltpu.SemaphoreType.DMA] * 8,
    )(td, ts, dstp, srcp)


_MROWS = NP // NS
_MZCH = _MROWS // CS
_TROWS = NPG // NS


def _scatter_body(valm_h, valt_h, dst_h, outm_h, outt_h,
                  idx0, idx1, idxt0, idxt1, vm0, vm1, vt0, vt1,
                  accm, acct, sl0, sl1):
    cid = lax.axis_index("c")
    sid = lax.axis_index("s")
    wid = sid * NC + cid
    base = wid * EWP
    mrow0 = sid * _MROWS
    trow0 = sid * _TROWS
    idx = (idx0, idx1)
    idxt = (idxt0, idxt1)
    vm = (vm0, vm1)
    vt = (vt0, vt1)
    sl = (sl0, sl1)

    zero16 = jnp.zeros((16,), _f32)

    @pl.loop(0, CS)
    def _zrow(i):
        for c in range(128 // 16):
            vm0[i, pl.ds(c * 16, 16)] = zero16

    @pl.loop(0, _MZCH)
    def _zaccm(k):
        pltpu.sync_copy(vm0, accm.at[pl.ds(mrow0 + k * CS, CS)])

    pltpu.sync_copy(vm0.at[pl.ds(0, _TROWS)], acct.at[pl.ds(trow0, _TROWS)])

    plsc.subcore_barrier()

    def start_loads(chunk, b):
        off = base + chunk * CS
        pltpu.async_copy(dst_h.at[pl.ds(off, CS)], idx[b], sl[b])
        pltpu.async_copy(valm_h.at[pl.ds(off, CS)], vm[b], sl[b])
        pltpu.async_copy(valt_h.at[pl.ds(off, CS)], vt[b], sl[b])

    def wait_loads(b):
        pltpu.make_async_copy(dst_h.at[pl.ds(base, CS)], idx[b], sl[b]).wait()
        pltpu.make_async_copy(valm_h.at[pl.ds(base, CS)], vm[b], sl[b]).wait()
        pltpu.make_async_copy(valt_h.at[pl.ds(base, CS)], vt[b], sl[b]).wait()

    start_loads(0, 0)

    @pl.loop(0, NCHS, step=2)
    def _pair(j):
        for b in range(2):
            cur = j + b
            nxt = cur + 1

            @pl.when(nxt < NCHS)
            def _():
                start_loads(nxt, 1 - b)

            wait_loads(b)
            for g in range(CS // 16):
                idxt[b][pl.ds(g * 16, 16)] = (
                    idx[b][pl.ds(g * 16, 16)] >> jnp.full((16,), 4, jnp.int32))
            pltpu.sync_copy(vm[b], accm.at[idx[b]], add=True)
            pltpu.sync_copy(vt[b], acct.at[idxt[b]], add=True)

    plsc.subcore_barrier()

    @pl.loop(0, _MZCH)
    def _woutm(k):
        pltpu.sync_copy(accm.at[pl.ds(mrow0 + k * CS, CS)],
                        outm_h.at[cid, pl.ds(mrow0 + k * CS, CS)])

    pltpu.sync_copy(acct.at[pl.ds(trow0, _TROWS)],
                    outt_h.at[cid, pl.ds(trow0, _TROWS)])


def _sc_scatter(vals_m, vals_t, dstp):
    return pl.kernel(
        _scatter_body,
        out_type=(
            jax.ShapeDtypeStruct((NC, NP, 128), _f32),
            jax.ShapeDtypeStruct((NC, NPG, 128), _f32),
        ),
        mesh=_sc_mesh(),
        scratch_types=[
            pltpu.VMEM((CS,), jnp.int32),
            pltpu.VMEM((CS,), jnp.int32),
            pltpu.VMEM((CS,), jnp.int32),
            pltpu.VMEM((CS,), jnp.int32),
            pltpu.VMEM((CS, 128), _f32),
            pltpu.VMEM((CS, 128), _f32),
            pltpu.VMEM((CS, 128), _f32),
            pltpu.VMEM((CS, 128), _f32),
            pltpu.VMEM_SHARED((NP, 128), _f32),
            pltpu.VMEM_SHARED((NPG, 128), _f32),
            pltpu.SemaphoreType.DMA,
            pltpu.SemaphoreType.DMA,
        ],
    )(vals_m, vals_t, dstp)


def _iota_f32(cols):
    return lax.broadcasted_iota(jnp.int32, (NB, cols), 1).astype(_f32)


def _com_first_body(c_ref, nd_ref, com_ref, acc):
    i = pl.program_id(0)

    @pl.when(i == 0)
    def _():
        acc[...] = jnp.zeros_like(acc)

    nd = nd_ref[...]
    gidf = nd[:, 0:1]
    m = nd[:, 1:2]
    cc = c_ref[...][:, 0:3]
    oh = (gidf == _iota_f32(G)).astype(_f32)
    vals = jnp.concatenate([m * cc, m, jnp.zeros((NB, 4), _f32)], axis=1)
    acc[...] += lax.dot_general(oh, vals, (((0,), (0,)), ((), ())),
                                preferred_element_type=_f32)

    @pl.when(i == NBLK - 1)
    def _():
        a = acc[...]
        com = a[:, 0:3] / jnp.clip(a[:, 3:4], 1e-6, None)
        com_ref[...] = jnp.concatenate([com, jnp.zeros((G, 5), _f32)], axis=1)


def _com_first(coords8, nodes8):
    return pl.pallas_call(
        _com_first_body,
        grid=(NBLK,),
        in_specs=[
            pl.BlockSpec((NB, 8), lambda i: (i, 0)),
            pl.BlockSpec((NB, 8), lambda i: (i, 0)),
        ],
        out_specs=pl.BlockSpec((G, 8), lambda i: (0, 0)),
        out_shape=jax.ShapeDtypeStruct((G, 8), _f32),
        scratch_shapes=[pltpu.VMEM((G, 8), _f32)],
    )(coords8, nodes8)


def _com_next_body(cz_ref, atA_ref, atB_ref, nd_ref, com_ref, cnew_ref, acc):
    i = pl.program_id(0)

    @pl.when(i == 0)
    def _():
        acc[...] = jnp.zeros_like(acc)

    nd = nd_ref[...]
    gidf = nd[:, 0:1]
    m = nd[:, 1:2]
    cnew = (cz_ref[...][:, 0:3]
            + atA_ref[...][:, 0:3] + atB_ref[...][:, 0:3])
    cnew_ref[...] = jnp.concatenate([cnew, jnp.zeros((NB, 5), _f32)], axis=1)
    oh = (gidf == _iota_f32(G)).astype(_f32)
    vals = jnp.concatenate([m * cnew, m, jnp.zeros((NB, 4), _f32)], axis=1)
    acc[...] += lax.dot_general(oh, vals, (((0,), (0,)), ((), ())),
                                preferred_element_type=_f32)

    @pl.when(i == NBLK - 1)
    def _():
        a = acc[...]
        com = a[:, 0:3] / jnp.clip(a[:, 3:4], 1e-6, None)
        com_ref[...] = jnp.concatenate([com, jnp.zeros((G, 5), _f32)], axis=1)


def _com_next(cz8, aggtA, aggtB, nodes8):
    return pl.pallas_call(
        _com_next_body,
        grid=(NBLK,),
        in_specs=[
            pl.BlockSpec((NB, 8), lambda i: (i, 0)),
            pl.BlockSpec((NB, 8), lambda i: (i, 0)),
            pl.BlockSpec((NB, 8), lambda i: (i, 0)),
            pl.BlockSpec((NB, 8), lambda i: (i, 0)),
        ],
        out_specs=[
            pl.BlockSpec((G, 8), lambda i: (0, 0)),
            pl.BlockSpec((NB, 8), lambda i: (i, 0)),
        ],
        out_shape=[
            jax.ShapeDtypeStruct((G, 8), _f32),
            jax.ShapeDtypeStruct((NP, 8), _f32),
        ],
        scratch_shapes=[pltpu.VMEM((G, 8), _f32)],
    )(cz8, aggtA, aggtB, nodes8)


def _pack_row(h, cz, craw):
    hb = lax.bitcast_convert_type(h, jnp.int32) + jnp.int32(0x8000)
    w = (lax.shift_right_logical(hb[:, 0:64], 16)
         | (hb[:, 64:128] & jnp.int32(-65536)))
    cb = lax.bitcast_convert_type(jnp.concatenate([cz, craw], axis=1),
                                  jnp.int32)
    return jnp.concatenate([w, cb, jnp.zeros((NB, 58), jnp.int32)], axis=1)


def _tables(h, cz, craw, Wd, Ws, eb1):
    td = _pack_row(jnp.dot(h, Wd, preferred_element_type=_f32) + eb1, cz, craw)
    ts = _pack_row(jnp.dot(h, Ws, preferred_element_type=_f32), -cz, -craw)
    return td, ts


def _init_body(nd_ref, craw_ref, f16_ref, com_ref, embp_ref, pWt_ref, pWb_ref,
               pb_ref, Wd_ref, Ws_ref, eb1_ref, h_ref, td_ref, ts_ref, cz_ref):
    nd = nd_ref[...]
    gidf = nd[:, 0:1]
    anf = nd[:, 2:3]
    ohA = (anf == _iota_f32(128)).astype(_f32)
    tbl = jnp.dot(embp_ref[...], pWt_ref[...], preferred_element_type=_f32)
    h = (jnp.dot(ohA, tbl, preferred_element_type=_f32)
         + jnp.dot(f16_ref[...], pWb_ref[...], preferred_element_type=_f32)
         + pb_ref[...])
    ohG = (gidf == _iota_f32(G)).astype(_f32)
    craw = craw_ref[...][:, 0:3]
    cz = craw - jnp.dot(ohG, com_ref[...], preferred_element_type=_f32)[:, 0:3]
    h_ref[...] = h
    td_ref[...], ts_ref[...] = _tables(h, cz, craw, Wd_ref[...], Ws_ref[...],
                                       eb1_ref[...])
    cz_ref[...] = jnp.concatenate([cz, jnp.zeros((NB, 5), _f32)], axis=1)


def _full(shp):
    return pl.BlockSpec(shp, lambda i: tuple(0 for _ in shp))


def _init(nodes8, coords8, feat16, com0, embp, pWt, pWb, pb, Wd, Ws, eb1):
    return pl.pallas_call(
        _init_body,
        grid=(NBLK,),
        in_specs=[
            pl.BlockSpec((NB, 8), lambda i: (i, 0)),
            pl.BlockSpec((NB, 8), lambda i: (i, 0)),
            pl.BlockSpec((NB, 16), lambda i: (i, 0)),
            _full((G, 8)),
            _full((128, 128)),
            _full((128, 128)),
            _full((16, 128)),
            _full((1, 128)),
            _full((128, 128)),
            _full((128, 128)),
            _full((1, 128)),
        ],
        out_specs=[
            pl.BlockSpec((NB, 128), lambda i: (i, 0)),
            pl.BlockSpec((NB, 128), lambda i: (i, 0)),
            pl.BlockSpec((NB, 128), lambda i: (i, 0)),
            pl.BlockSpec((NB, 8), lambda i: (i, 0)),
        ],
        out_shape=[
            jax.ShapeDtypeStruct((NP, 128), _f32),
            jax.ShapeDtypeStruct((NP, 128), jnp.int32),
            jax.ShapeDtypeStruct((NP, 128), jnp.int32),
            jax.ShapeDtypeStruct((NP, 8), _f32),
        ],
    )(nodes8, coords8, feat16, com0, embp, pWt, pWb, pb, Wd, Ws, eb1)


def _unpack_h(g):
    lo = lax.bitcast_convert_type(lax.shift_left(g[:, 0:64], 16), _f32)
    hi = lax.bitcast_convert_type(g[:, 0:64] & jnp.int32(-65536), _f32)
    return jnp.concatenate([lo, hi], axis=1)


def _unpack_pair(gd, gs):
    hsum = _unpack_h(gd) + _unpack_h(gs)
    cd = lax.bitcast_convert_type(gd[:, 64:70], _f32)
    cs = lax.bitcast_convert_type(gs[:, 64:70], _f32)
    diff = cd[:, 0:3] + cs[:, 0:3]
    d0 = cd[:, 3:6] + cs[:, 3:6]
    return hsum, diff, d0


def _edge_core(hsum, diff, a, dstv, wa, eW2, eb2, xW1, xb1, xW2p):
    v = hsum + a * wa
    m1 = jax.nn.silu(v)
    m = jax.nn.silu(jnp.dot(m1, eW2, preferred_element_type=_f32) + eb2)
    t = jax.nn.silu(jnp.dot(m, xW1, preferred_element_type=_f32) + xb1)
    w = jnp.dot(t, xW2p, preferred_element_type=_f32)[:, 0:1]
    dist = jnp.sqrt(jnp.sum(diff * diff, axis=1, keepdims=True) + 1e-8)
    trans = diff / (dist + 1.0) * w
    off = (dstv & 15) * 8
    lanes = lax.broadcasted_iota(jnp.int32, (hsum.shape[0], 128), 1)
    packed = (jnp.where(lanes == off, trans[:, 0:1], 0.0)
              + jnp.where(lanes == off + 1, trans[:, 1:2], 0.0)
              + jnp.where(lanes == off + 2, trans[:, 2:3], 0.0))
    return m, packed


def _edge_a1_body(gd_ref, gs_ref, dst_ref, wa_ref, eW2_ref, eb2_ref, xW1_ref,
                  xb1_ref, xW2_ref, vm_ref, vt_ref, a_ref):
    hsum, diff, d0 = _unpack_pair(gd_ref[...], gs_ref[...])
    a = jnp.sum(d0 * d0, axis=1, keepdims=True)
    vm_ref[...], vt_ref[...] = _edge_core(
        hsum, diff, a, dst_ref[...], wa_ref[...], eW2_ref[...], eb2_ref[...],
        xW1_ref[...], xb1_ref[...], xW2_ref[...])
    a_ref[...] = a


def _edge_a2_body(gd_ref, gs_ref, dst_ref, a_in_ref, wa_ref, eW2_ref, eb2_ref,
                  xW1_ref, xb1_ref, xW2_ref, vm_ref, vt_ref):
    hsum, diff, _ = _unpack_pair(gd_ref[...], gs_ref[...])
    vm_ref[...], vt_ref[...] = _edge_core(
        hsum, diff, a_in_ref[...], dst_ref[...], wa_ref[...],
        eW2_ref[...], eb2_ref[...], xW1_ref[...], xb1_ref[...], xW2_ref[...])


def _edge_weight_specs():
    return [_full((1, 128)), _full((128, 128)), _full((1, 128)),
            _full((128, 128)), _full((1, 128)), _full((128, 8))]


_EDGE_OUT_MT = [
    pl.BlockSpec((TE, 128), lambda i: (i, 0)),
    pl.BlockSpec((TE, 128), lambda i: (i, 0)),
]


def _edge_a1(gd, gs, dst2, wa, eW2, eb2, xW1, xb1, xW2p):
    return pl.pallas_call(
        _edge_a1_body,
        grid=(NEB,),
        in_specs=[
            pl.BlockSpec((TE, 128), lambda i: (i, 0)),
            pl.BlockSpec((TE, 128), lambda i: (i, 0)),
            pl.BlockSpec((TE, 1), lambda i: (i, 0)),
        ] + _edge_weight_specs(),
        out_specs=_EDGE_OUT_MT + [pl.BlockSpec((TE, 1), lambda i: (i, 0))],
        out_shape=[
            jax.ShapeDtypeStruct((EP, 128), _f32),
            jax.ShapeDtypeStruct((EP, 128), _f32),
            jax.ShapeDtypeStruct((EP, 1), _f32),
        ],
    )(gd, gs, dst2, wa, eW2, eb2, xW1, xb1, xW2p)


def _edge_a2(gd, gs, dst2, a_edges, wa, eW2, eb2, xW1, xb1, xW2p):
    return pl.pallas_call(
        _edge_a2_body,
        grid=(NEB,),
        in_specs=[
            pl.BlockSpec((TE, 128), lambda i: (i, 0)),
            pl.BlockSpec((TE, 128), lambda i: (i, 0)),
            pl.BlockSpec((TE, 1), lambda i: (i, 0)),
            pl.BlockSpec((TE, 1), lambda i: (i, 0)),
        ] + _edge_weight_specs(),
        out_specs=_EDGE_OUT_MT,
        out_shape=[
            jax.ShapeDtypeStruct((EP, 128), _f32),
            jax.ShapeDtypeStruct((EP, 128), _f32),
        ],
    )(gd, gs, dst2, a_edges, wa, eW2, eb2, xW1, xb1, xW2p)


def _b2_body(h_ref, amA_ref, amB_ref, cnew_ref, nd_ref, com_ref, hW1a_ref,
             hW1b_ref, hb1_ref, hW2_ref, hb2_ref, Wd_ref, Ws_ref, eb1_ref,
             hn_ref, td_ref, ts_ref, cz_ref):
    h = h_ref[...]
    aggm = amA_ref[...] + amB_ref[...]
    u = jax.nn.silu(jnp.dot(h, hW1a_ref[...], preferred_element_type=_f32)
                    + jnp.dot(aggm, hW1b_ref[...], preferred_element_type=_f32)
                    + hb1_ref[...])
    hn = h + jnp.dot(u, hW2_ref[...], preferred_element_type=_f32) + hb2_ref[...]
    nd = nd_ref[...]
    gidf = nd[:, 0:1]
    ohG = (gidf == _iota_f32(G)).astype(_f32)
    cz = (cnew_ref[...][:, 0:3]
          - jnp.dot(ohG, com_ref[...], preferred_element_type=_f32)[:, 0:3])
    hn_ref[...] = hn
    zcraw = jnp.zeros((NB, 3), _f32)
    td_ref[...], ts_ref[...] = _tables(hn, cz, zcraw, Wd_ref[...], Ws_ref[...],
                                       eb1_ref[...])
    cz_ref[...] = jnp.concatenate([cz, jnp.zeros((NB, 5), _f32)], axis=1)


def _b2(h, amA, amB, cnew8, nodes8, com, hW1a, hW1b, hb1, hW2, hb2, Wd, Ws,
        eb1):
    return pl.pallas_call(
        _b2_body,
        grid=(NBLK,),
        in_specs=[
            pl.BlockSpec((NB, 128), lambda i: (i, 0)),
            pl.BlockSpec((NB, 128), lambda i: (i, 0)),
            pl.BlockSpec((NB, 128), lambda i: (i, 0)),
            pl.BlockSpec((NB, 8), lambda i: (i, 0)),
            pl.BlockSpec((NB, 8), lambda i: (i, 0)),
            _full((G, 8)),
            _full((128, 128)), _full((128, 128)), _full((1, 128)),
            _full((128, 128)), _full((1, 128)),
            _full((128, 128)), _full((128, 128)), _full((1, 128)),
        ],
        out_specs=[
            pl.BlockSpec((NB, 128), lambda i: (i, 0)),
            pl.BlockSpec((NB, 128), lambda i: (i, 0)),
            pl.BlockSpec((NB, 128), lambda i: (i, 0)),
            pl.BlockSpec((NB, 8), lambda i: (i, 0)),
        ],
        out_shape=[
            jax.ShapeDtypeStruct((NP, 128), _f32),
            jax.ShapeDtypeStruct((NP, 128), jnp.int32),
            jax.ShapeDtypeStruct((NP, 128), jnp.int32),
            jax.ShapeDtypeStruct((NP, 8), _f32),
        ],
    )(h, amA, amB, cnew8, nodes8, com, hW1a, hW1b, hb1, hW2, hb2, Wd, Ws, eb1)


def _final_body(cnew_ref, nd_ref, com_ref, out_ref):
    nd = nd_ref[...]
    gidf = nd[:, 0:1]
    ohG = (gidf == _iota_f32(G)).astype(_f32)
    cz = (cnew_ref[...][:, 0:3]
          - jnp.dot(ohG, com_ref[...], preferred_element_type=_f32)[:, 0:3])
    out_ref[...] = jnp.concatenate([cz, jnp.zeros((NB, 5), _f32)], axis=1)


def _final(cnew8, nodes8, com):
    return pl.pallas_call(
        _final_body,
        grid=(NBLK,),
        in_specs=[
            pl.BlockSpec((NB, 8), lambda i: (i, 0)),
            pl.BlockSpec((NB, 8), lambda i: (i, 0)),
            _full((G, 8)),
        ],
        out_specs=pl.BlockSpec((NB, 8), lambda i: (i, 0)),
        out_shape=jax.ShapeDtypeStruct((NP, 8), _f32),
    )(cnew8, nodes8, com)


def _pad_edges(idx):
    per = idx.reshape(NW, E // NW)
    per = jnp.pad(per, ((0, 0), (0, EWP - E // NW)), constant_values=N)
    return per.reshape(EP).astype(jnp.int32)


def kernel(temb, masses, masses_normalized, cond_labels, cond_mask, moments,
           coords, emb, proj_W, proj_b, eW1, eb1, eW2, eb2, xW1, xb1, xW2,
           hW1, hb1, hW2, hb2, atom_nums, edge_index, node_graph_idx):
    srcp = _pad_edges(edge_index[0])
    dstp = _pad_edges(edge_index[1])
    dst2 = dstp[:, None]

    gidf = node_graph_idx.astype(_f32)[:, None]
    anf = atom_nums.astype(_f32)[:, None]
    nodes8 = jnp.pad(
        jnp.concatenate([gidf, masses, anf], axis=1), ((0, NP - N), (0, 5)))
    coords8 = jnp.pad(coords, ((0, NP - N), (0, 5)))
    feat16 = jnp.pad(
        jnp.concatenate([temb, masses, masses_normalized, cond_labels,
                         cond_mask, moments], axis=1), ((0, NP - N), (0, 4)))
    embp = jnp.pad(emb, ((0, 128 - emb.shape[0]), (0, 0)))
    pWt = proj_W[:AF]
    pWb = jnp.pad(proj_W[AF:], ((0, 4), (0, 0)))
    pb = proj_b[None, :]
    xW2p = jnp.pad(xW2, ((0, 0), (0, 0), (0, 7)))

    com0 = _com_first(coords8, nodes8)
    h, td, ts, cz8 = _init(nodes8, coords8, feat16, com0, embp, pWt, pWb, pb,
                           eW1[0, 0:H], eW1[0, H:2 * H], eb1[0][None, :])

    a_edges = None
    out = None
    for i in range(LAYERS):
        gd, gs = _sc_gather(td, ts, dstp, srcp)
        wa = eW1[i, 2 * H:2 * H + 1]
        if i == 0:
            vals_m, vals_t, a_edges = _edge_a1(
                gd, gs, dst2, wa, eW2[i], eb2[i][None, :], xW1[i],
                xb1[i][None, :], xW2p[i])
        else:
            vals_m, vals_t = _edge_a2(
                gd, gs, dst2, a_edges, wa, eW2[i], eb2[i][None, :], xW1[i],
                xb1[i][None, :], xW2p[i])
        out_m, out_t = _sc_scatter(vals_m, vals_t, dstp)
        aggtA = out_t[0].reshape(NP, 8)
        aggtB = out_t[1].reshape(NP, 8)
        com, cnew8 = _com_next(cz8, aggtA, aggtB, nodes8)
        if i + 1 < LAYERS:
            h, td, ts, cz8 = _b2(h, out_m[0], out_m[1], cnew8, nodes8, com,
                                 hW1[i, 0:H], hW1[i, H:2 * H],
                                 hb1[i][None, :], hW2[i], hb2[i][None, :],
                                 eW1[i + 1, 0:H], eW1[i + 1, H:2 * H],
                                 eb1[i + 1][None, :])
        else:
            out = _final(cnew8, nodes8, com)
    return out[:N, 0:3]
